# baseline (speedup 1.0000x reference)
import functools

import jax
import jax.numpy as jnp
from jax import lax
from jax.experimental import pallas as pl
from jax.experimental.pallas import tpu as pltpu
from jax.experimental.pallas import tpu_sc as plsc

JITTER_EPS = 0.01


def _router_body(x_ref, gwt_ref, i1_ref, i2_ref, m1_ref, m2_ref):
    x = x_ref[...]
    s = jnp.dot(x, gwt_ref[...], preferred_element_type=jnp.float32)
    E = s.shape[-1]
    neg_inf = jnp.float32(-jnp.inf)
    m1 = jnp.max(s, axis=-1, keepdims=True)
    i1 = jnp.argmax(s, axis=-1).astype(jnp.int32)
    f1 = jnp.maximum(jnp.abs(s), m1)
    l1 = jnp.where((m1 - s) / f1 > 2.0 * JITTER_EPS, neg_inf, s)
    mult1 = 1.0 / jnp.sum(jnp.exp(l1 - m1), axis=-1)
    cols = lax.broadcasted_iota(jnp.int32, s.shape, 1)
    s2 = jnp.where(cols == i1[:, None], neg_inf, s)
    m2 = jnp.max(s2, axis=-1, keepdims=True)
    i2 = jnp.argmax(s2, axis=-1).astype(jnp.int32)
    f2 = jnp.maximum(jnp.abs(s), m2)
    l2 = jnp.where((m2 - s) / f2 > 2.0 * JITTER_EPS, neg_inf, s2)
    mult2 = 1.0 / jnp.sum(jnp.exp(l2 - m2), axis=-1)
    i1_ref[...] = i1
    i2_ref[...] = i2
    m1_ref[...] = mult1
    m2_ref[...] = mult2


def _run_router(x, gate_w, T, H, E, RBT):
    grid = (T // RBT,)
    return pl.pallas_call(
        _router_body,
        grid=grid,
        in_specs=[
            pl.BlockSpec((RBT, H), lambda b: (b, 0)),
            pl.BlockSpec((H, E), lambda b: (0, 0)),
        ],
        out_specs=[
            pl.BlockSpec((RBT,), lambda b: (b,)),
            pl.BlockSpec((RBT,), lambda b: (b,)),
            pl.BlockSpec((RBT,), lambda b: (b,)),
            pl.BlockSpec((RBT,), lambda b: (b,)),
        ],
        out_shape=[
            jax.ShapeDtypeStruct((T,), jnp.int32),
            jax.ShapeDtypeStruct((T,), jnp.int32),
            jax.ShapeDtypeStruct((T,), jnp.float32),
            jax.ShapeDtypeStruct((T,), jnp.float32),
        ],
    )(x, gate_w.T)


def _dispatch_metadata(i1, i2, mult1, mult2, T, E, BT, NB, P):
    e_flat = jnp.concatenate([i1, i2])
    w_flat = jnp.concatenate([mult1, mult2])
    order = jnp.argsort(e_flat)
    rank_flat = jnp.argsort(order)
    counts = jnp.bincount(e_flat, length=E).astype(jnp.int32)
    csum = jnp.cumsum(counts)
    offsets = csum - counts
    padded_counts = ((counts + BT - 1) // BT) * BT
    padded_cum = jnp.cumsum(padded_counts)
    padded_offsets = padded_cum - padded_counts
    p_idx = jnp.arange(P, dtype=jnp.int32)
    e_p = jnp.searchsorted(padded_cum, p_idx, side='right').astype(jnp.int32)
    e_p = jnp.minimum(e_p, E - 1)
    r_p = p_idx - padded_offsets[e_p]
    is_real = r_p < counts[e_p]
    j_p = jnp.where(is_real, offsets[e_p] + r_p, 0)
    pair_p = order[j_p]
    row_tok = jnp.where(is_real, (pair_p % T).astype(jnp.int32), 0)
    w_row = jnp.where(is_real, w_flat[pair_p], 0.0)
    inv_pos = (padded_offsets[e_flat] + (rank_flat - offsets[e_flat])
               ).astype(jnp.int32)
    block_starts = jnp.arange(NB, dtype=jnp.int32) * BT
    total_padded = padded_cum[-1]
    valid = (block_starts < total_padded).astype(jnp.int32)
    be = jnp.searchsorted(padded_cum, block_starts, side='right').astype(jnp.int32)
    nvalid = total_padded // BT
    be_last = be[nvalid - 1]
    be = jnp.where(valid == 1, be, be_last)
    return row_tok, w_row, inv_pos[:T], inv_pos[T:], be, valid


def _make_gather(P, H, dtype):
    info = plsc.get_sparse_core_info()
    NC, NS = info.num_cores, info.num_subcores
    NW = NC * NS
    pw = P // NW
    CH = 64
    assert pw % CH == 0
    nch = pw // CH
    mesh = plsc.VectorSubcoreMesh(core_axis_name="c", subcore_axis_name="s")

    @functools.partial(
        pl.kernel, mesh=mesh,
        out_type=jax.ShapeDtypeStruct((P, H), dtype),
        scratch_types=[
            pltpu.VMEM((pw,), jnp.int32),
            pltpu.VMEM((CH, H), dtype),
            pltpu.VMEM((CH, H), dtype),
            pltpu.SemaphoreType.DMA,
            pltpu.SemaphoreType.DMA,
        ],
    )
    def gk(x_hbm, idx_hbm, out_hbm, idx_v, rows0, rows1, sem0, sem1):
        wid = lax.axis_index("s") * NC + lax.axis_index("c")
        base = wid * pw
        pltpu.sync_copy(idx_hbm.at[pl.ds(base, pw)], idx_v)
        bufs = (rows0, rows1)
        sems = (sem0, sem1)
        cps = [None] * nch
        cps[0] = pltpu.async_copy(
            x_hbm.at[idx_v.at[pl.ds(0, CH)]], bufs[0], sems[0])
        for ch in range(nch):
            if ch + 1 < nch:
                cps[ch + 1] = pltpu.async_copy(
                    x_hbm.at[idx_v.at[pl.ds((ch + 1) * CH, CH)]],
                    bufs[(ch + 1) % 2], sems[(ch + 1) % 2])
            cps[ch].wait()
            pltpu.sync_copy(bufs[ch % 2], out_hbm.at[pl.ds(base + ch * CH, CH)])

    return gk


def _make_ffn(P, H, F, BT, FT, NB, NF):
    def body(be_ref, valid_ref, xs_ref, w1_ref, w3_ref, w2_ref, wr_ref,
             out_ref, acc_ref, w1b_ref, w3b_ref, w2b_ref):
        f = pl.program_id(0)
        b = pl.program_id(1)
        new_w = (b == 0) | (be_ref[b] != be_ref[jnp.maximum(b - 1, 0)])

        @pl.when(valid_ref[b] == 1)
        def _():
            @pl.when(new_w)
            def _():
                w1b_ref[...] = w1_ref[0].astype(jnp.bfloat16)
                w3b_ref[...] = w3_ref[0].astype(jnp.bfloat16)
                w2b_ref[...] = w2_ref[0].astype(jnp.bfloat16)

            xs = xs_ref[...]
            a = jnp.dot(xs, w1b_ref[...], preferred_element_type=jnp.float32)
            c = jnp.dot(xs, w3b_ref[...], preferred_element_type=jnp.float32)
            h = (a * jax.nn.sigmoid(a)) * c
            y = jnp.dot(h.astype(jnp.bfloat16), w2b_ref[...],
                        preferred_element_type=jnp.float32)

            @pl.when(f == 0)
            def _():
                acc_ref[b] = y

            @pl.when(f != 0)
            def _():
                acc_ref[b] += y

            @pl.when(f == NF - 1)
            def _():
                out_ref[...] = acc_ref[b] * wr_ref[...]

        @pl.when((valid_ref[b] == 0) & (f == NF - 1))
        def _():
            out_ref[...] = jnp.zeros_like(out_ref)

    grid_spec = pltpu.PrefetchScalarGridSpec(
        num_scalar_prefetch=2,
        grid=(NF, NB),
        in_specs=[
            pl.BlockSpec((BT, H), lambda f, b, be, va: (b, 0)),
            pl.BlockSpec((1, H, FT), lambda f, b, be, va: (be[b], 0, f)),
            pl.BlockSpec((1, H, FT), lambda f, b, be, va: (be[b], 0, f)),
            pl.BlockSpec((1, FT, H), lambda f, b, be, va: (be[b], f, 0)),
            pl.BlockSpec((BT, 1), lambda f, b, be, va: (b, 0)),
        ],
        out_specs=pl.BlockSpec(
            (BT, H), lambda f, b, be, va: (jnp.where(f == NF - 1, b, 0), 0)),
        scratch_shapes=[
            pltpu.VMEM((NB, BT, H), jnp.float32),
            pltpu.VMEM((H, FT), jnp.bfloat16),
            pltpu.VMEM((H, FT), jnp.bfloat16),
            pltpu.VMEM((FT, H), jnp.bfloat16),
        ],
    )
    return pl.pallas_call(
        body,
        grid_spec=grid_spec,
        out_shape=jax.ShapeDtypeStruct((P, H), jnp.float32),
        compiler_params=pltpu.CompilerParams(
            dimension_semantics=("arbitrary", "arbitrary")),
    )


def _make_combine(P, H, T):
    info = plsc.get_sparse_core_info()
    NC, NS = info.num_cores, info.num_subcores
    NW = NC * NS
    tw = T // NW
    CH = 32
    assert tw % CH == 0
    nlane = H // 16
    mesh = plsc.VectorSubcoreMesh(core_axis_name="c", subcore_axis_name="s")

    @functools.partial(
        pl.kernel, mesh=mesh,
        out_type=jax.ShapeDtypeStruct((T, H), jnp.float32),
        scratch_types=[
            pltpu.VMEM((tw,), jnp.int32),
            pltpu.VMEM((tw,), jnp.int32),
            pltpu.VMEM((CH, H), jnp.float32),
            pltpu.VMEM((CH, H), jnp.float32),
            pltpu.SemaphoreType.DMA,
            pltpu.SemaphoreType.DMA,
        ],
    )
    def ck(y_hbm, inv0_hbm, inv1_hbm, out_hbm, i0_v, i1_v, bufa, bufb, sema, semb):
        wid = lax.axis_index("s") * NC + lax.axis_index("c")
        base = wid * tw
        pltpu.sync_copy(inv0_hbm.at[pl.ds(base, tw)], i0_v)
        pltpu.sync_copy(inv1_hbm.at[pl.ds(base, tw)], i1_v)
        for ch in range(tw // CH):
            ca = pltpu.async_copy(
                y_hbm.at[i0_v.at[pl.ds(ch * CH, CH)]], bufa, sema)
            cb = pltpu.async_copy(
                y_hbm.at[i1_v.at[pl.ds(ch * CH, CH)]], bufb, semb)
            ca.wait()
            cb.wait()

            def row_body(r, carry):
                def lane_body(c, carry2):
                    sl = pl.ds(c * 16, 16)
                    bufa[r, sl] = bufa[r, sl] + bufb[r, sl]
                    return carry2
                return lax.fori_loop(0, nlane, lane_body, carry, unroll=8)

            lax.fori_loop(0, CH, row_body, 0)
            pltpu.sync_copy(bufa, out_hbm.at[pl.ds(base + ch * CH, CH)])

    return ck


def kernel(hidden_states, gate_w, w1, w2, w3):
    B, S, H = hidden_states.shape
    E, _, F = w1.shape
    T = B * S
    x = hidden_states.reshape(T, H)

    BT = 256
    FT = 512
    NB = (2 * T) // BT + E
    P = NB * BT
    NF = F // FT
    RBT = 256

    i1, i2, mult1, mult2 = _run_router(x, gate_w, T, H, E, RBT)
    row_tok, w_row, inv0, inv1, be, valid = _dispatch_metadata(
        i1, i2, mult1, mult2, T, E, BT, NB, P)

    x_bf = x.astype(jnp.bfloat16)
    x_pack = jax.lax.bitcast_convert_type(
        x_bf.reshape(T, H // 2, 2), jnp.uint32)
    xs_pack = _make_gather(P, H // 2, jnp.uint32)(x_pack, row_tok)
    x_sorted = jax.lax.bitcast_convert_type(
        xs_pack, jnp.bfloat16).reshape(P, H)
    y_sorted = _make_ffn(P, H, F, BT, FT, NB, NF)(
        be, valid, x_sorted, w1, w3, w2, w_row.reshape(P, 1))
    out = _make_combine(P, H, T)(y_sorted, inv0, inv1)
    return out.reshape(B, S, H)


# speedup vs baseline: 1.3480x; 1.3480x over previous
import functools

import jax
import jax.numpy as jnp
from jax import lax
from jax.experimental import pallas as pl
from jax.experimental.pallas import tpu as pltpu
from jax.experimental.pallas import tpu_sc as plsc

JITTER_EPS = 0.01


def _router_body(x_ref, gwt_ref, i1_ref, i2_ref, m1_ref, m2_ref):
    x = x_ref[...]
    s = jnp.dot(x, gwt_ref[...], preferred_element_type=jnp.float32)
    E = s.shape[-1]
    neg_inf = jnp.float32(-jnp.inf)
    m1 = jnp.max(s, axis=-1, keepdims=True)
    i1 = jnp.argmax(s, axis=-1).astype(jnp.int32)
    f1 = jnp.maximum(jnp.abs(s), m1)
    l1 = jnp.where((m1 - s) / f1 > 2.0 * JITTER_EPS, neg_inf, s)
    mult1 = 1.0 / jnp.sum(jnp.exp(l1 - m1), axis=-1)
    cols = lax.broadcasted_iota(jnp.int32, s.shape, 1)
    s2 = jnp.where(cols == i1[:, None], neg_inf, s)
    m2 = jnp.max(s2, axis=-1, keepdims=True)
    i2 = jnp.argmax(s2, axis=-1).astype(jnp.int32)
    f2 = jnp.maximum(jnp.abs(s), m2)
    l2 = jnp.where((m2 - s) / f2 > 2.0 * JITTER_EPS, neg_inf, s2)
    mult2 = 1.0 / jnp.sum(jnp.exp(l2 - m2), axis=-1)
    i1_ref[...] = i1
    i2_ref[...] = i2
    m1_ref[...] = mult1
    m2_ref[...] = mult2


def _run_router(x, gate_w, T, H, E, RBT):
    grid = (T // RBT,)
    return pl.pallas_call(
        _router_body,
        grid=grid,
        in_specs=[
            pl.BlockSpec((RBT, H), lambda b: (b, 0)),
            pl.BlockSpec((H, E), lambda b: (0, 0)),
        ],
        out_specs=[
            pl.BlockSpec((RBT,), lambda b: (b,)),
            pl.BlockSpec((RBT,), lambda b: (b,)),
            pl.BlockSpec((RBT,), lambda b: (b,)),
            pl.BlockSpec((RBT,), lambda b: (b,)),
        ],
        out_shape=[
            jax.ShapeDtypeStruct((T,), jnp.int32),
            jax.ShapeDtypeStruct((T,), jnp.int32),
            jax.ShapeDtypeStruct((T,), jnp.float32),
            jax.ShapeDtypeStruct((T,), jnp.float32),
        ],
    )(x, gate_w.T)


def _sel8(idx, table):
    eids = jnp.arange(table.shape[0], dtype=jnp.int32)
    return jnp.sum(
        jnp.where(idx[:, None] == eids[None, :], table[None, :], 0),
        axis=1, dtype=table.dtype)


def _dispatch_metadata(i1, i2, mult1, mult2, T, E, BT, NB, P):
    T2 = 2 * T
    e_flat = jnp.concatenate([i1, i2])
    w_flat = jnp.concatenate([mult1, mult2])
    pair_iota = jnp.arange(T2, dtype=jnp.int32)
    _, order, w_sorted = lax.sort((e_flat, pair_iota, w_flat), num_keys=1)
    tok_sorted = jnp.where(order >= T, order - T, order)
    rank_flat = jnp.argsort(order).astype(jnp.int32)
    eids = jnp.arange(E, dtype=jnp.int32)
    counts = jnp.sum(e_flat[:, None] == eids[None, :], axis=0,
                     dtype=jnp.int32)
    csum = jnp.cumsum(counts)
    offsets = csum - counts
    padded_counts = ((counts + BT - 1) // BT) * BT
    padded_cum = jnp.cumsum(padded_counts)
    padded_offsets = padded_cum - padded_counts
    inv_pos = (_sel8(e_flat, padded_offsets) + rank_flat
               - _sel8(e_flat, offsets)).astype(jnp.int32)
    block_starts = jnp.arange(NB, dtype=jnp.int32) * BT
    total_padded = padded_cum[-1]
    valid = (block_starts < total_padded).astype(jnp.int32)
    be = jnp.sum(block_starts[:, None] >= padded_cum[None, :], axis=1,
                 dtype=jnp.int32)
    nvalid = total_padded // BT
    be_last = be[nvalid - 1]
    be = jnp.where(valid == 1, jnp.minimum(be, E - 1), be_last)
    p_idx = jnp.arange(P, dtype=jnp.int32)
    slot_e = jnp.minimum(
        jnp.sum(p_idx[:, None] >= padded_cum[None, :], axis=1,
                dtype=jnp.int32), E - 1)
    r_p = p_idx - _sel8(slot_e, padded_offsets.astype(jnp.int32))
    is_real = r_p < _sel8(slot_e, counts)
    delta = (padded_offsets - offsets).astype(jnp.int32)
    zpad = jnp.zeros((P,), jnp.int32)
    tok_ext = jnp.concatenate([zpad, tok_sorted, zpad[:P - T2]])
    w_ext = jnp.concatenate(
        [zpad.astype(jnp.float32), w_sorted, jnp.zeros((P - T2,), jnp.float32)])
    row_tok = jnp.zeros((P,), jnp.int32)
    w_row = jnp.zeros((P,), jnp.float32)
    for e in range(E):
        sh_tok = lax.dynamic_slice(tok_ext, (P - delta[e],), (P,))
        sh_w = lax.dynamic_slice(w_ext, (P - delta[e],), (P,))
        sel = slot_e == e
        row_tok = jnp.where(sel, sh_tok, row_tok)
        w_row = jnp.where(sel, sh_w, w_row)
    row_tok = jnp.where(is_real, row_tok, 0)
    w_row = jnp.where(is_real, w_row, 0.0)
    return row_tok, w_row, inv_pos[:T], inv_pos[T:], be, valid


def _make_gather(P, H):
    info = plsc.get_sparse_core_info()
    NC, NS = info.num_cores, info.num_subcores
    NW = NC * NS
    pw = P // NW
    CH = 48
    assert pw % CH == 0 and CH % 8 == 0
    nch = pw // CH
    mesh = plsc.VectorSubcoreMesh(core_axis_name="c", subcore_axis_name="s")

    @functools.partial(
        pl.kernel, mesh=mesh,
        out_type=jax.ShapeDtypeStruct((P, H), jnp.float32),
        scratch_types=[pltpu.VMEM((CH,), jnp.int32) for _ in range(nch)] + [
            pltpu.VMEM((CH, H), jnp.float32),
            pltpu.VMEM((CH, H), jnp.float32),
            pltpu.SemaphoreType.DMA,
            pltpu.SemaphoreType.DMA,
        ],
    )
    def gk(x_hbm, idx_hbm, xs_hbm, *refs):
        idx_bufs = refs[:nch]
        rows = refs[nch:nch + 2]
        sems = refs[nch + 2:]
        wid = lax.axis_index("s") * NC + lax.axis_index("c")
        base = wid * pw
        for ch in range(nch):
            pltpu.sync_copy(idx_hbm.at[pl.ds(base + ch * CH, CH)],
                            idx_bufs[ch])
        cps = [None] * nch
        cps[0] = pltpu.async_copy(x_hbm.at[idx_bufs[0]], rows[0], sems[0])
        for ch in range(nch):
            if ch + 1 < nch:
                cps[ch + 1] = pltpu.async_copy(
                    x_hbm.at[idx_bufs[ch + 1]], rows[(ch + 1) % 2],
                    sems[(ch + 1) % 2])
            cps[ch].wait()
            pltpu.sync_copy(rows[ch % 2],
                            xs_hbm.at[pl.ds(base + ch * CH, CH)])

    return gk


def _make_ffn(P, H, F, BT, FT, NB, NF):
    def body(be_ref, valid_ref, xs_ref, w1_ref, w3_ref, w2_ref, wr_ref,
             out_ref, acc_ref, w1b_ref, w3b_ref, w2b_ref):
        f = pl.program_id(0)
        b = pl.program_id(1)
        new_w = (b == 0) | (be_ref[b] != be_ref[jnp.maximum(b - 1, 0)])

        @pl.when(valid_ref[b] == 1)
        def _():
            @pl.when(new_w)
            def _():
                w1b_ref[...] = w1_ref[0].astype(jnp.bfloat16)
                w3b_ref[...] = w3_ref[0].astype(jnp.bfloat16)
                w2b_ref[...] = w2_ref[0].astype(jnp.bfloat16)

            xs = xs_ref[...].astype(jnp.bfloat16)
            a = jnp.dot(xs, w1b_ref[...], preferred_element_type=jnp.float32)
            c = jnp.dot(xs, w3b_ref[...], preferred_element_type=jnp.float32)
            h = (a * jax.nn.sigmoid(a)) * c
            y = jnp.dot(h.astype(jnp.bfloat16), w2b_ref[...],
                        preferred_element_type=jnp.float32)

            @pl.when(f == 0)
            def _():
                acc_ref[b] = y

            @pl.when(f != 0)
            def _():
                acc_ref[b] += y

            @pl.when(f == NF - 1)
            def _():
                out_ref[...] = acc_ref[b] * wr_ref[...]

        @pl.when((valid_ref[b] == 0) & (f == NF - 1))
        def _():
            out_ref[...] = jnp.zeros_like(out_ref)

    grid_spec = pltpu.PrefetchScalarGridSpec(
        num_scalar_prefetch=2,
        grid=(NF, NB),
        in_specs=[
            pl.BlockSpec((BT, H), lambda f, b, be, va: (b, 0)),
            pl.BlockSpec((1, H, FT), lambda f, b, be, va: (be[b], 0, f)),
            pl.BlockSpec((1, H, FT), lambda f, b, be, va: (be[b], 0, f)),
            pl.BlockSpec((1, FT, H), lambda f, b, be, va: (be[b], f, 0)),
            pl.BlockSpec((BT, 1), lambda f, b, be, va: (b, 0)),
        ],
        out_specs=pl.BlockSpec(
            (BT, H), lambda f, b, be, va: (jnp.where(f == NF - 1, b, 0), 0)),
        scratch_shapes=[
            pltpu.VMEM((NB, BT, H), jnp.float32),
            pltpu.VMEM((H, FT), jnp.bfloat16),
            pltpu.VMEM((H, FT), jnp.bfloat16),
            pltpu.VMEM((FT, H), jnp.bfloat16),
        ],
    )
    return pl.pallas_call(
        body,
        grid_spec=grid_spec,
        out_shape=jax.ShapeDtypeStruct((P, H), jnp.float32),
        compiler_params=pltpu.CompilerParams(
            dimension_semantics=("arbitrary", "arbitrary")),
    )


def _make_combine(P, H, T):
    info = plsc.get_sparse_core_info()
    NC, NS = info.num_cores, info.num_subcores
    NW = NC * NS
    tw = T // NW
    CH = 32
    assert tw % CH == 0
    nlane = H // 16
    mesh = plsc.VectorSubcoreMesh(core_axis_name="c", subcore_axis_name="s")

    nch = tw // CH

    @functools.partial(
        pl.kernel, mesh=mesh,
        out_type=jax.ShapeDtypeStruct((T, H), jnp.float32),
        scratch_types=[pltpu.VMEM((CH,), jnp.int32) for _ in range(2 * nch)] + [
            pltpu.VMEM((CH, H), jnp.float32),
            pltpu.VMEM((CH, H), jnp.float32),
            pltpu.SemaphoreType.DMA,
            pltpu.SemaphoreType.DMA,
        ],
    )
    def ck(y_hbm, inv0_hbm, inv1_hbm, out_hbm, *refs):
        i0_bufs = refs[:nch]
        i1_bufs = refs[nch:2 * nch]
        bufa, bufb, sema, semb = refs[2 * nch:]
        wid = lax.axis_index("s") * NC + lax.axis_index("c")
        base = wid * tw
        for ch in range(nch):
            pltpu.sync_copy(inv0_hbm.at[pl.ds(base + ch * CH, CH)], i0_bufs[ch])
            pltpu.sync_copy(inv1_hbm.at[pl.ds(base + ch * CH, CH)], i1_bufs[ch])
        for ch in range(nch):
            ca = pltpu.async_copy(y_hbm.at[i0_bufs[ch]], bufa, sema)
            cb = pltpu.async_copy(y_hbm.at[i1_bufs[ch]], bufb, semb)
            ca.wait()
            cb.wait()

            def row_body(r, carry):
                def lane_body(c, carry2):
                    sl = pl.ds(c * 16, 16)
                    bufa[r, sl] = bufa[r, sl] + bufb[r, sl]
                    return carry2
                return lax.fori_loop(0, nlane, lane_body, carry, unroll=8)

            lax.fori_loop(0, CH, row_body, 0)
            pltpu.sync_copy(bufa, out_hbm.at[pl.ds(base + ch * CH, CH)])

    return ck


def kernel(hidden_states, gate_w, w1, w2, w3):
    B, S, H = hidden_states.shape
    E, _, F = w1.shape
    T = B * S
    x = hidden_states.reshape(T, H)

    BT = 256
    FT = 512
    NB = (2 * T) // BT + E
    P = NB * BT
    NF = F // FT
    RBT = 256

    i1, i2, mult1, mult2 = _run_router(x, gate_w, T, H, E, RBT)
    row_tok, w_row, inv0, inv1, be, valid = _dispatch_metadata(
        i1, i2, mult1, mult2, T, E, BT, NB, P)

    x_sorted = _make_gather(P, H)(x, row_tok)
    y_sorted = _make_ffn(P, H, F, BT, FT, NB, NF)(
        be, valid, x_sorted, w1, w3, w2, w_row.reshape(P, 1))
    out = _make_combine(P, H, T)(y_sorted, inv0, inv1)
    return out.reshape(B, S, H)


# speedup vs baseline: 1.4123x; 1.0477x over previous
import functools

import jax
import jax.numpy as jnp
from jax import lax
from jax.experimental import pallas as pl
from jax.experimental.pallas import tpu as pltpu
from jax.experimental.pallas import tpu_sc as plsc

JITTER_EPS = 0.01


def _router_body(x_ref, gwt_ref, i1_ref, i2_ref, m1_ref, m2_ref):
    x = x_ref[...]
    s = jnp.dot(x, gwt_ref[...], preferred_element_type=jnp.float32)
    E = s.shape[-1]
    neg_inf = jnp.float32(-jnp.inf)
    m1 = jnp.max(s, axis=-1, keepdims=True)
    i1 = jnp.argmax(s, axis=-1).astype(jnp.int32)
    f1 = jnp.maximum(jnp.abs(s), m1)
    l1 = jnp.where((m1 - s) / f1 > 2.0 * JITTER_EPS, neg_inf, s)
    mult1 = 1.0 / jnp.sum(jnp.exp(l1 - m1), axis=-1)
    cols = lax.broadcasted_iota(jnp.int32, s.shape, 1)
    s2 = jnp.where(cols == i1[:, None], neg_inf, s)
    m2 = jnp.max(s2, axis=-1, keepdims=True)
    i2 = jnp.argmax(s2, axis=-1).astype(jnp.int32)
    f2 = jnp.maximum(jnp.abs(s), m2)
    l2 = jnp.where((m2 - s) / f2 > 2.0 * JITTER_EPS, neg_inf, s2)
    mult2 = 1.0 / jnp.sum(jnp.exp(l2 - m2), axis=-1)
    i1_ref[...] = i1
    i2_ref[...] = i2
    m1_ref[...] = mult1
    m2_ref[...] = mult2


def _run_router(x, gate_w, T, H, E, RBT):
    grid = (T // RBT,)
    return pl.pallas_call(
        _router_body,
        grid=grid,
        in_specs=[
            pl.BlockSpec((RBT, H), lambda b: (b, 0)),
            pl.BlockSpec((H, E), lambda b: (0, 0)),
        ],
        out_specs=[
            pl.BlockSpec((RBT,), lambda b: (b,)),
            pl.BlockSpec((RBT,), lambda b: (b,)),
            pl.BlockSpec((RBT,), lambda b: (b,)),
            pl.BlockSpec((RBT,), lambda b: (b,)),
        ],
        out_shape=[
            jax.ShapeDtypeStruct((T,), jnp.int32),
            jax.ShapeDtypeStruct((T,), jnp.int32),
            jax.ShapeDtypeStruct((T,), jnp.float32),
            jax.ShapeDtypeStruct((T,), jnp.float32),
        ],
    )(x, gate_w.T)


def _sel8(idx, table):
    eids = jnp.arange(table.shape[0], dtype=jnp.int32)
    return jnp.sum(
        jnp.where(idx[:, None] == eids[None, :], table[None, :], 0),
        axis=1, dtype=table.dtype)


def _dispatch_metadata(i1, i2, mult1, mult2, T, E, BT, NB, P):
    T2 = 2 * T
    e_flat = jnp.concatenate([i1, i2])
    w_flat = jnp.concatenate([mult1, mult2])
    pair_iota = jnp.arange(T2, dtype=jnp.int32)
    _, order, w_sorted = lax.sort((e_flat, pair_iota, w_flat), num_keys=1)
    tok_sorted = jnp.where(order >= T, order - T, order)
    rank_flat = jnp.argsort(order).astype(jnp.int32)
    eids = jnp.arange(E, dtype=jnp.int32)
    counts = jnp.sum(e_flat[:, None] == eids[None, :], axis=0,
                     dtype=jnp.int32)
    csum = jnp.cumsum(counts)
    offsets = csum - counts
    padded_counts = ((counts + BT - 1) // BT) * BT
    padded_cum = jnp.cumsum(padded_counts)
    padded_offsets = padded_cum - padded_counts
    inv_pos = (_sel8(e_flat, padded_offsets) + rank_flat
               - _sel8(e_flat, offsets)).astype(jnp.int32)
    block_starts = jnp.arange(NB, dtype=jnp.int32) * BT
    total_padded = padded_cum[-1]
    valid = (block_starts < total_padded).astype(jnp.int32)
    be = jnp.sum(block_starts[:, None] >= padded_cum[None, :], axis=1,
                 dtype=jnp.int32)
    nvalid = total_padded // BT
    be_last = be[nvalid - 1]
    be = jnp.where(valid == 1, jnp.minimum(be, E - 1), be_last)
    p_idx = jnp.arange(P, dtype=jnp.int32)
    slot_e = jnp.minimum(
        jnp.sum(p_idx[:, None] >= padded_cum[None, :], axis=1,
                dtype=jnp.int32), E - 1)
    r_p = p_idx - _sel8(slot_e, padded_offsets.astype(jnp.int32))
    is_real = r_p < _sel8(slot_e, counts)
    delta = (padded_offsets - offsets).astype(jnp.int32)
    zpad = jnp.zeros((P,), jnp.int32)
    tok_ext = jnp.concatenate([zpad, tok_sorted, zpad[:P - T2]])
    w_ext = jnp.concatenate(
        [zpad.astype(jnp.float32), w_sorted, jnp.zeros((P - T2,), jnp.float32)])
    row_tok = jnp.zeros((P,), jnp.int32)
    w_row = jnp.zeros((P,), jnp.float32)
    for e in range(E):
        sh_tok = lax.dynamic_slice(tok_ext, (P - delta[e],), (P,))
        sh_w = lax.dynamic_slice(w_ext, (P - delta[e],), (P,))
        sel = slot_e == e
        row_tok = jnp.where(sel, sh_tok, row_tok)
        w_row = jnp.where(sel, sh_w, w_row)
    row_tok = jnp.where(is_real, row_tok, 0)
    w_row = jnp.where(is_real, w_row, 0.0)
    return row_tok, w_row, inv_pos[:T], inv_pos[T:], be, valid


def _make_gather(P, HW):
    info = plsc.get_sparse_core_info()
    NC, NS = info.num_cores, info.num_subcores
    NW = NC * NS
    pw = P // NW
    CH = 48
    SUB = 16
    assert pw % CH == 0 and CH % SUB == 0
    nch = pw // CH
    nsub = CH // SUB
    mesh = plsc.VectorSubcoreMesh(core_axis_name="c", subcore_axis_name="s")

    @functools.partial(
        pl.kernel, mesh=mesh,
        out_type=jax.ShapeDtypeStruct((P, HW), jnp.float32),
        scratch_types=[pltpu.VMEM((CH,), jnp.int32) for _ in range(nch)] + [
            pltpu.VMEM((CH, HW), jnp.float32),
            pltpu.VMEM((CH, HW), jnp.float32),
            pltpu.SemaphoreType.DMA,
            pltpu.SemaphoreType.DMA,
        ],
    )
    def gk(x_hbm, idx_hbm, xs_hbm, *refs):
        idx_bufs = refs[:nch]
        rows = refs[nch:nch + 2]
        sems = refs[nch + 2:]
        wid = lax.axis_index("s") * NC + lax.axis_index("c")
        base = wid * pw
        for ch in range(nch):
            pltpu.sync_copy(idx_hbm.at[pl.ds(base + ch * CH, CH)],
                            idx_bufs[ch])

        def fire(ch):
            buf = rows[ch % 2]
            sem = sems[ch % 2]
            return [
                pltpu.async_copy(
                    x_hbm.at[idx_bufs[ch].at[pl.ds(k * SUB, SUB)]],
                    buf.at[pl.ds(k * SUB, SUB)], sem)
                for k in range(nsub)
            ]

        cps = [None] * nch
        cps[0] = fire(0)
        for ch in range(nch):
            if ch + 1 < nch:
                cps[ch + 1] = fire(ch + 1)
            for cp in cps[ch]:
                cp.wait()
            pltpu.sync_copy(rows[ch % 2],
                            xs_hbm.at[pl.ds(base + ch * CH, CH)])

    return gk


def _make_ffn(P, H, F, BT, FT, NB, NF):
    def body(be_ref, valid_ref, xs_ref, w1_ref, w3_ref, w2_ref, wr_ref,
             out_ref, acc_ref, xsb_ref, w1b_ref, w3b_ref, w2b_ref):
        f = pl.program_id(0)
        b = pl.program_id(1)
        new_w = (b == 0) | (be_ref[b] != be_ref[jnp.maximum(b - 1, 0)])

        @pl.when(valid_ref[b] == 1)
        def _():
            @pl.when(new_w)
            def _():
                w1b_ref[...] = w1_ref[0].astype(jnp.bfloat16)
                w3b_ref[...] = w3_ref[0].astype(jnp.bfloat16)
                w2b_ref[...] = w2_ref[0].astype(jnp.bfloat16)

            @pl.when(f == 0)
            def _():
                xsb_ref[b] = xs_ref[...].astype(jnp.bfloat16)

            xs = xsb_ref[b]
            a = jnp.dot(xs, w1b_ref[...], preferred_element_type=jnp.float32)
            c = jnp.dot(xs, w3b_ref[...], preferred_element_type=jnp.float32)
            h = (a * jax.nn.sigmoid(a)) * c
            y = jnp.dot(h.astype(jnp.bfloat16), w2b_ref[...],
                        preferred_element_type=jnp.float32)

            @pl.when(f == 0)
            def _():
                acc_ref[b] = y

            @pl.when((f != 0) & (f != NF - 1))
            def _():
                acc_ref[b] += y

            @pl.when(f == NF - 1)
            def _():
                out_ref[...] = (acc_ref[b] + y) * wr_ref[...]

        @pl.when((valid_ref[b] == 0) & (f == NF - 1))
        def _():
            out_ref[...] = jnp.zeros_like(out_ref)

    grid_spec = pltpu.PrefetchScalarGridSpec(
        num_scalar_prefetch=2,
        grid=(NF, NB),
        in_specs=[
            pl.BlockSpec((BT, H), lambda f, b, be, va: (jnp.where(f == 0, b, 0), 0)),
            pl.BlockSpec((1, H, FT), lambda f, b, be, va: (be[b], 0, f)),
            pl.BlockSpec((1, H, FT), lambda f, b, be, va: (be[b], 0, f)),
            pl.BlockSpec((1, FT, H), lambda f, b, be, va: (be[b], f, 0)),
            pl.BlockSpec((BT, 1), lambda f, b, be, va: (b, 0)),
        ],
        out_specs=pl.BlockSpec(
            (BT, H), lambda f, b, be, va: (jnp.where(f == NF - 1, b, 0), 0)),
        scratch_shapes=[
            pltpu.VMEM((NB, BT, H), jnp.float32),
            pltpu.VMEM((NB, BT, H), jnp.bfloat16),
            pltpu.VMEM((H, FT), jnp.bfloat16),
            pltpu.VMEM((H, FT), jnp.bfloat16),
            pltpu.VMEM((FT, H), jnp.bfloat16),
        ],
    )
    return pl.pallas_call(
        body,
        grid_spec=grid_spec,
        out_shape=jax.ShapeDtypeStruct((P, H), jnp.float32),
        compiler_params=pltpu.CompilerParams(
            dimension_semantics=("arbitrary", "arbitrary")),
    )


def _make_combine(P, H, T):
    info = plsc.get_sparse_core_info()
    NC, NS = info.num_cores, info.num_subcores
    NW = NC * NS
    tw = T // NW
    CH = 32
    assert tw % CH == 0
    nlane = H // 16
    mesh = plsc.VectorSubcoreMesh(core_axis_name="c", subcore_axis_name="s")

    nch = tw // CH

    @functools.partial(
        pl.kernel, mesh=mesh,
        out_type=jax.ShapeDtypeStruct((T, H), jnp.float32),
        scratch_types=[pltpu.VMEM((CH,), jnp.int32) for _ in range(2 * nch)] + [
            pltpu.VMEM((CH, H), jnp.float32),
            pltpu.VMEM((CH, H), jnp.float32),
            pltpu.SemaphoreType.DMA,
            pltpu.SemaphoreType.DMA,
        ],
    )
    def ck(y_hbm, inv0_hbm, inv1_hbm, out_hbm, *refs):
        i0_bufs = refs[:nch]
        i1_bufs = refs[nch:2 * nch]
        bufa, bufb, sema, semb = refs[2 * nch:]
        wid = lax.axis_index("s") * NC + lax.axis_index("c")
        base = wid * tw
        for ch in range(nch):
            pltpu.sync_copy(inv0_hbm.at[pl.ds(base + ch * CH, CH)], i0_bufs[ch])
            pltpu.sync_copy(inv1_hbm.at[pl.ds(base + ch * CH, CH)], i1_bufs[ch])
        SUB = 16
        for ch in range(nch):
            cps = []
            for k in range(CH // SUB):
                sl = pl.ds(k * SUB, SUB)
                cps.append(pltpu.async_copy(
                    y_hbm.at[i0_bufs[ch].at[sl]], bufa.at[sl], sema))
                cps.append(pltpu.async_copy(
                    y_hbm.at[i1_bufs[ch].at[sl]], bufb.at[sl], semb))
            for cp in cps:
                cp.wait()

            def row_body(r, carry):
                def lane_body(c, carry2):
                    sl = pl.ds(c * 16, 16)
                    bufa[r, sl] = bufa[r, sl] + bufb[r, sl]
                    return carry2
                return lax.fori_loop(0, nlane, lane_body, carry, unroll=8)

            lax.fori_loop(0, CH, row_body, 0)
            pltpu.sync_copy(bufa, out_hbm.at[pl.ds(base + ch * CH, CH)])

    return ck


def kernel(hidden_states, gate_w, w1, w2, w3):
    B, S, H = hidden_states.shape
    E, _, F = w1.shape
    T = B * S
    x = hidden_states.reshape(T, H)

    BT = 256
    FT = 512
    NB = (2 * T) // BT + E
    P = NB * BT
    NF = F // FT
    RBT = 256

    i1, i2, mult1, mult2 = _run_router(x, gate_w, T, H, E, RBT)
    row_tok, w_row, inv0, inv1, be, valid = _dispatch_metadata(
        i1, i2, mult1, mult2, T, E, BT, NB, P)

    x_sorted = _make_gather(P, H)(x, row_tok)
    y_sorted = _make_ffn(P, H, F, BT, FT, NB, NF)(
        be, valid, x_sorted, w1, w3, w2, w_row.reshape(P, 1))
    out = _make_combine(P, H, T)(y_sorted, inv0, inv1)
    return out.reshape(B, S, H)
